# Initial kernel scaffold; baseline (speedup 1.0000x reference)
import jax
import jax.numpy as jnp
from jax.experimental import pallas as pl


def kernel(emb, adj, H_v):
    raise NotImplementedError("write your pallas kernel here")


# speedup vs baseline: 1.9982x; 1.9982x over previous
import functools

import jax
import jax.numpy as jnp
from jax.experimental import pallas as pl
from jax.experimental.pallas import tpu as pltpu

N = 4096
D = 64
BM = 512


def _body(adj_ref, emb_ref, hv_ref, out_ref, we_ref, w_ref):
    @pl.when(pl.program_id(0) == 0)
    def _init():
        s = jnp.dot(emb_ref[...], hv_ref[...],
                    preferred_element_type=jnp.float32)
        w = jnp.exp(s)
        w_ref[...] = w
        we_ref[...] = emb_ref[...] * w

    a = adj_ref[...]
    num = jnp.dot(a, we_ref[...], preferred_element_type=jnp.float32)
    den = jnp.dot(a, w_ref[...], preferred_element_type=jnp.float32)
    out_ref[...] = num / den


@jax.jit
def kernel(emb, adj, H_v):
    return pl.pallas_call(
        _body,
        grid=(N // BM,),
        in_specs=[
            pl.BlockSpec((BM, N), lambda i: (i, 0)),
            pl.BlockSpec((N, D), lambda i: (0, 0)),
            pl.BlockSpec((D, 1), lambda i: (0, 0)),
        ],
        out_specs=pl.BlockSpec((BM, D), lambda i: (i, 0)),
        out_shape=jax.ShapeDtypeStruct((N, D), jnp.float32),
        scratch_shapes=[
            pltpu.VMEM((N, D), jnp.float32),
            pltpu.VMEM((N, 1), jnp.float32),
        ],
    )(adj, emb, H_v)


# speedup vs baseline: 2.2622x; 1.1321x over previous
import functools

import jax
import jax.numpy as jnp
from jax.experimental import pallas as pl
from jax.experimental.pallas import tpu as pltpu

N = 4096
D = 64
BM = 512


def _body(adj_ref, emb_ref, hv_ref, out_ref, we2_ref):
    @pl.when(pl.program_id(0) == 0)
    def _init():
        s = jnp.dot(emb_ref[...], hv_ref[...],
                    preferred_element_type=jnp.float32)
        w = jnp.exp(s)
        we2_ref[:, :D] = emb_ref[...] * w
        we2_ref[:, D:] = jnp.broadcast_to(w, (N, D))

    res = jnp.dot(adj_ref[...], we2_ref[...],
                  preferred_element_type=jnp.float32)
    out_ref[...] = res[:, :D] / res[:, D:D + 1]


@jax.jit
def kernel(emb, adj, H_v):
    return pl.pallas_call(
        _body,
        grid=(N // BM,),
        in_specs=[
            pl.BlockSpec((BM, N), lambda i: (i, 0)),
            pl.BlockSpec((N, D), lambda i: (0, 0)),
            pl.BlockSpec((D, 1), lambda i: (0, 0)),
        ],
        out_specs=pl.BlockSpec((BM, D), lambda i: (i, 0)),
        out_shape=jax.ShapeDtypeStruct((N, D), jnp.float32),
        scratch_shapes=[
            pltpu.VMEM((N, 2 * D), jnp.float32),
        ],
    )(adj, emb, H_v)
